# baseline (speedup 1.0000x reference)
import jax
import jax.numpy as jnp
from jax import lax
from jax.experimental import pallas as pl
from jax.experimental.pallas import tpu as pltpu
from jax.experimental.pallas import tpu_sc as plsc

B = 16384
K = 16
NC = 2
NS = 16
NW = NC * NS
BPW = B // NW
CHUNK = 128
NCHUNK = BPW // CHUNK
GROUPS = BPW // K
NROW = 1000000
IPW = K * NCHUNK


def _bpr_body(user_h, pos_h, neg_h, eu_h, ei_h, out_h,
              idx_u, idx_p, idx_n, u_col, p_col, n_col, out_v, sem):
    wid = lax.axis_index("s") * NC + lax.axis_index("c")
    irow0 = wid * IPW

    pltpu.sync_copy(user_h.at[pl.ds(irow0, IPW)], idx_u)
    pltpu.sync_copy(pos_h.at[pl.ds(irow0, IPW)], idx_p)
    pltpu.sync_copy(neg_h.at[pl.ds(irow0, IPW)], idx_n)

    copies = []
    for k in range(K):
        for j in range(NCHUNK):
            row = k * NCHUNK + j
            dst = pl.ds(k * BPW + j * CHUNK, CHUNK)
            copies.append(pltpu.async_copy(
                eu_h.at[plsc.Indices(idx_u.at[row])], u_col.at[dst], sem))
            copies.append(pltpu.async_copy(
                ei_h.at[plsc.Indices(idx_p.at[row])], p_col.at[dst], sem))
            copies.append(pltpu.async_copy(
                ei_h.at[plsc.Indices(idx_n.at[row])], n_col.at[dst], sem))
    for c in copies:
        c.wait()

    def group(g, carry):
        acc = jnp.zeros((K,), jnp.float32)
        for k in range(K):
            sl = pl.ds(k * BPW + g * K, K)
            acc = acc + u_col[sl] * (p_col[sl] - n_col[sl])
        out_v[pl.ds(g * K, K)] = acc
        return carry

    lax.fori_loop(0, GROUPS, group, 0)

    pltpu.sync_copy(out_v, out_h.at[pl.ds(wid * BPW, BPW)])


@jax.jit
def kernel(user, pos_item, neg_item, embedding_user, embedding_item):
    mesh = plsc.VectorSubcoreMesh(core_axis_name="c", subcore_axis_name="s")
    f = pl.kernel(
        _bpr_body,
        out_type=jax.ShapeDtypeStruct((B,), jnp.float32),
        mesh=mesh,
        scratch_types=[
            pltpu.VMEM((IPW, CHUNK), jnp.int32),
            pltpu.VMEM((IPW, CHUNK), jnp.int32),
            pltpu.VMEM((IPW, CHUNK), jnp.int32),
            pltpu.VMEM((K * BPW,), jnp.float32),
            pltpu.VMEM((K * BPW,), jnp.float32),
            pltpu.VMEM((K * BPW,), jnp.float32),
            pltpu.VMEM((BPW,), jnp.float32),
            pltpu.SemaphoreType.DMA,
        ],
        compiler_params=pltpu.CompilerParams(
            needs_layout_passes=False,
            use_tc_tiling_on_sc=False,
        ),
    )

    koff = (jnp.arange(K, dtype=jnp.int32) * NROW).reshape(1, K, 1, 1)

    def expand(ix):
        ix4 = ix.astype(jnp.int32).reshape(NW, 1, NCHUNK, CHUNK)
        return jnp.broadcast_to(ix4 + koff, (NW, K, NCHUNK, CHUNK)).reshape(
            NW * IPW, CHUNK)

    u2 = expand(user)
    p2 = expand(pos_item)
    n2 = expand(neg_item)
    eu_t = embedding_user.T.reshape(K * NROW)
    ei_t = embedding_item.T.reshape(K * NROW)
    return f(u2, p2, n2, eu_t, ei_t)
